# baseline (speedup 1.0000x reference)
import functools

import jax
import jax.numpy as jnp
import numpy as np
from jax import lax
from jax.experimental import pallas as pl
from jax.experimental.pallas import tpu as pltpu
from jax.experimental.pallas import tpu_sc as plsc

H = 768
NLANE = 16
NCOL = H // NLANE
N_PAGES = 16
SEQ = 512
BODY = 502
NPT = 10
GRID = 14
IMG_SIDE = 224
NVIS = GRID * GRID
ROW_OUT = BODY + NVIS + NPT


def _pe_table():
    n, d = 4, H
    pos = np.arange(n)[:, None].astype(np.float32)
    i = np.arange(d)[None, :].astype(np.float32)
    angle = pos / np.power(10000.0, (2.0 * np.floor(i / 2.0)) / d)
    pe = np.zeros((n, d), dtype=np.float32)
    pe[:, 0::2] = np.sin(angle[:, 0::2])
    pe[:, 1::2] = np.cos(angle[:, 1::2])
    return pe


def _grid_idx():
    xs = (np.arange(GRID) * 1000) // GRID
    xe = ((np.arange(GRID) + 1) * 1000) // GRID
    pad = lambda a: np.pad(a, (0, 16 - GRID)).astype(np.int32)
    return pad(xs), pad(xe)


def _visual_matmul(patches, W, b):
    def body(a_ref, w_ref, b_ref, o_ref):
        o_ref[...] = jnp.dot(a_ref[...], w_ref[...],
                             preferred_element_type=jnp.float32) + b_ref[...]

    return pl.pallas_call(
        body,
        grid=(8,),
        in_specs=[pl.BlockSpec((392, 256), lambda i: (i, 0)),
                  pl.BlockSpec((256, H), lambda i: (0, 0)),
                  pl.BlockSpec((1, H), lambda i: (0, 0))],
        out_specs=pl.BlockSpec((392, H), lambda i: (i, 0)),
        out_shape=jax.ShapeDtypeStruct((N_PAGES * NVIS, H), jnp.float32),
    )(patches, W, b.reshape(1, H))


def _sc_embed(shared, x_t, y_t, vd, pe, ids, i1, i2, i3, i4, xs_i, xe_i):
    mesh = plsc.VectorSubcoreMesh(core_axis_name="c", subcore_axis_name="s")

    @functools.partial(
        pl.kernel, mesh=mesh,
        out_type=jax.ShapeDtypeStruct((N_PAGES * ROW_OUT, H), jnp.float32),
        compiler_params=pltpu.CompilerParams(use_tc_tiling_on_sc=False),
        scratch_types=[
            pltpu.VMEM((256,), jnp.int32),
            pltpu.VMEM((256,), jnp.int32),
            pltpu.VMEM((256,), jnp.int32),
            pltpu.VMEM((256,), jnp.int32),
            pltpu.VMEM((256,), jnp.int32),
            pltpu.VMEM((16,), jnp.int32),
            pltpu.VMEM((16,), jnp.int32),
            pltpu.VMEM((16, H), jnp.float32),
            pltpu.VMEM((16, H), jnp.float32),
            pltpu.VMEM((16, H), jnp.float32),
            pltpu.VMEM((16, H), jnp.float32),
            pltpu.VMEM((16, H), jnp.float32),
            pltpu.VMEM((16, H), jnp.float32),
            pltpu.VMEM((16, H), jnp.float32),
            pltpu.VMEM((16, H), jnp.float32),
            pltpu.VMEM((16, H), jnp.float32),
            pltpu.VMEM((1, H), jnp.float32),
            pltpu.SemaphoreType.DMA,
            pltpu.SemaphoreType.DMA,
        ],
    )
    def k(shared_h, xt_h, yt_h, vd_h, pe_h, ids_h, i1_h, i2_h, i3_h, i4_h,
          xs_h, xe_h, out_h,
          ids_v, i1_v, i2_v, i3_v, i4_v, xs_v, xe_v,
          b0, b1, b2, b3, b4, bxs, bys, bxe, bye, pe_v,
          sem_a, sem_c):
        n = lax.axis_index("s")
        h = lax.axis_index("c")
        tok0 = n * SEQ + h * 256
        out0 = n * ROW_OUT + h * 256

        pltpu.sync_copy(ids_h.at[pl.ds(tok0, 256)], ids_v)
        pltpu.sync_copy(i1_h.at[pl.ds(tok0, 256)], i1_v)
        pltpu.sync_copy(i2_h.at[pl.ds(tok0, 256)], i2_v)
        pltpu.sync_copy(i3_h.at[pl.ds(tok0, 256)], i3_v)
        pltpu.sync_copy(i4_h.at[pl.ds(tok0, 256)], i4_v)
        pltpu.sync_copy(xs_h, xs_v)
        pltpu.sync_copy(xe_h, xe_v)
        pltpu.sync_copy(pe_h.at[pl.ds(n % 4, 1)], pe_v)

        vg0 = pltpu.async_copy(xt_h.at[xs_v], bxs, sem_c)
        vg1 = pltpu.async_copy(yt_h.at[xs_v], bys, sem_c)
        vg2 = pltpu.async_copy(xt_h.at[xe_v], bxe, sem_c)
        vg3 = pltpu.async_copy(yt_h.at[xe_v], bye, sem_c)

        def _sum5(dst_r, rows):
            for g in range(NCOL):
                sl = pl.ds(g * NLANE, NLANE)
                ref0, r0 = rows[0]
                acc = ref0[r0, sl]
                for ref, rr in rows[1:]:
                    acc = acc + ref[rr, sl]
                b0[dst_r, sl] = acc

        def _token_chunk(ci):
            off = ci * 16
            g0 = pltpu.async_copy(shared_h.at[ids_v[pl.ds(off, 16)]], b0, sem_a)
            g1 = pltpu.async_copy(xt_h.at[i1_v[pl.ds(off, 16)]], b1, sem_a)
            g2 = pltpu.async_copy(yt_h.at[i2_v[pl.ds(off, 16)]], b2, sem_a)
            g3 = pltpu.async_copy(xt_h.at[i3_v[pl.ds(off, 16)]], b3, sem_a)
            g4 = pltpu.async_copy(yt_h.at[i4_v[pl.ds(off, 16)]], b4, sem_a)
            g0.wait(); g1.wait(); g2.wait(); g3.wait(); g4.wait()

            def rbody(r, carry):
                _sum5(r, [(b0, r), (b1, r), (b2, r), (b3, r), (b4, r)])
                return carry
            lax.fori_loop(0, 16, rbody, 0)

        def chunk_a(ci, carry):
            _token_chunk(ci)
            pltpu.sync_copy(b0, out_h.at[pl.ds(out0 + ci * 16, 16)])
            return carry
        lax.fori_loop(0, 15, chunk_a, 0)

        _token_chunk(15)

        @pl.when(h == 0)
        def _():
            pltpu.sync_copy(b0, out_h.at[pl.ds(out0 + 240, 16)])

        @pl.when(h == 1)
        def _():
            def peb(r, carry):
                for g in range(NCOL):
                    sl = pl.ds(g * NLANE, NLANE)
                    b0[r, sl] = b0[r, sl] + pe_v[0, sl]
                return carry
            lax.fori_loop(6, 16, peb, 0)
            pltpu.sync_copy(b0.at[pl.ds(0, 6)],
                            out_h.at[pl.ds(n * ROW_OUT + 496, 6)])
            pltpu.sync_copy(b0.at[pl.ds(6, 10)],
                            out_h.at[pl.ds(n * ROW_OUT + BODY + NVIS, 10)])

        vg0.wait(); vg1.wait(); vg2.wait(); vg3.wait()
        vbase = n * NVIS + h * 98
        obase = n * ROW_OUT + BODY + h * 98

        def _vis_rows(nrows, rg0):
            def vb(kk, carry):
                rg = rg0 + kk
                ii = rg // GRID
                jj = rg - ii * GRID
                _sum5(kk, [(b0, kk), (bxs, jj), (bys, ii), (bxe, jj), (bye, ii)])
                return carry
            lax.fori_loop(0, nrows, vb, 0)

        def chunk_c(ci, carry):
            off = ci * 16
            dcp = pltpu.async_copy(vd_h.at[pl.ds(vbase + off, 16)], b0, sem_a)
            dcp.wait()
            _vis_rows(16, h * 98 + off)
            pltpu.sync_copy(b0, out_h.at[pl.ds(obase + off, 16)])
            return carry
        lax.fori_loop(0, 6, chunk_c, 0)

        dcp = pltpu.async_copy(vd_h.at[pl.ds(vbase + 96, 2)],
                               b0.at[pl.ds(0, 2)], sem_a)
        dcp.wait()
        _vis_rows(2, h * 98 + 96)
        pltpu.sync_copy(b0.at[pl.ds(0, 2)], out_h.at[pl.ds(obase + 96, 2)])

    return k(shared, x_t, y_t, vd, pe, ids, i1, i2, i3, i4, xs_i, xe_i)


def kernel(input_ids, boxes, images, shared, x_table, y_table, W_vis, b_vis):
    ids = input_ids.reshape(-1).astype(jnp.int32)
    bf = boxes.reshape(-1, 4).astype(jnp.int32)
    i1, i2, i3, i4 = bf[:, 0], bf[:, 1], bf[:, 2], bf[:, 3]
    imgs = images.reshape(-1, IMG_SIDE, IMG_SIDE)
    patches = imgs.reshape(-1, GRID, 16, GRID, 16).transpose(0, 1, 3, 2, 4)
    patches = patches.reshape(N_PAGES * NVIS, 256)
    vd = _visual_matmul(patches, W_vis, b_vis)
    pe = jnp.asarray(_pe_table())
    xs, xe = _grid_idx()
    out = _sc_embed(shared, x_table, y_table, vd, pe, ids, i1, i2, i3, i4,
                    jnp.asarray(xs), jnp.asarray(xe))
    return out.reshape(N_PAGES, ROW_OUT, H)
